# baseline (speedup 1.0000x reference)
import functools

import jax
import jax.numpy as jnp
from jax.experimental import pallas as pl
from jax.experimental.pallas import tpu as pltpu

SUB = 8
SUBW = 1024


def _rotl(x, d):
    return (x << jnp.uint32(d)) | (x >> jnp.uint32(32 - d))


def _threefry_bits(j):
    ks0 = jnp.uint32(0)
    ks1 = jnp.uint32(1)
    ks2 = jnp.uint32(0x1BD11BDA) ^ ks0 ^ ks1
    ks = (ks0, ks1, ks2)
    rotations = ((13, 15, 26, 6), (17, 29, 16, 24))
    x0 = jnp.zeros_like(j) + ks0
    x1 = j + ks1
    for i in range(5):
        for r in rotations[i % 2]:
            x0 = x0 + x1
            x1 = _rotl(x1, r)
            x1 = x1 ^ x0
        x0 = x0 + ks[(i + 1) % 3]
        x1 = x1 + ks[(i + 2) % 3] + jnp.uint32(i + 1)
    return x0 ^ x1


def _iotas(shape, C):
    sub = jax.lax.broadcasted_iota(jnp.int32, shape, 0)
    lane = jax.lax.broadcasted_iota(jnp.int32, shape, 1)
    return sub * C + lane


RP = 2


def _row_body(temps_ref, logits_ref, probs_ref, tok_ref, *, V, C):
    toks = []
    for rp in range(RP):
        _one_row(temps_ref, logits_ref, probs_ref, toks, rp, V=V, C=C)
    tok_ref[...] = jnp.reshape(jnp.stack(toks), (RP, 1, 1))


def _one_row(temps_ref, logits_ref, probs_ref, toks, rp, *, V, C):
    r = pl.program_id(0) * RP + rp
    t_raw = temps_ref[r]
    t = jnp.where(t_raw < 1e-5, jnp.float32(1.0), t_raw)
    rt = jnp.float32(1.0) / t
    neginf = jnp.float32(-jnp.inf)
    big = jnp.int32(V)

    nfull = C // SUBW
    tailw = C - nfull * SUBW
    base_flat = _iotas((SUB, SUBW), C)
    tail_flat = _iotas((SUB, tailw), C) + nfull * SUBW if tailw else None

    vm = jnp.full((SUB, SUBW), neginf, jnp.float32)
    vg = jnp.full((SUB, SUBW), big, jnp.int32)
    for k in range(nfull):
        sc = logits_ref[rp, :, k * SUBW:(k + 1) * SUBW] * rt
        upd = sc > vm
        vm = jnp.maximum(vm, sc)
        vg = jnp.where(upd, base_flat + k * SUBW, vg)
    m = jnp.max(vm)
    g = jnp.min(jnp.where(vm == m, vg, big))
    if tailw:
        sc = logits_ref[rp, :, nfull * SUBW:C] * rt
        mt = jnp.max(sc)
        gt = jnp.min(jnp.where(sc == mt, tail_flat, big))
        g = jnp.where(mt > m, gt, g)
        m = jnp.maximum(m, mt)

    vs = jnp.zeros((SUB, SUBW), jnp.float32)
    for k in range(nfull):
        vs = vs + jnp.exp(logits_ref[rp, :, k * SUBW:(k + 1) * SUBW] * rt - m)
    s = jnp.sum(vs)
    if tailw:
        s = s + jnp.sum(jnp.exp(logits_ref[rp, :, nfull * SUBW:C] * rt - m))
    rs = jnp.float32(1.0) / s

    def chunk_ratio(x, flat0):
        e = jnp.exp(x * rt - m)
        probs = e * rs
        j = flat0.astype(jnp.uint32)
        bits = _threefry_bits(j)
        uf = jax.lax.bitcast_convert_type(
            (bits >> jnp.uint32(9)) | jnp.uint32(0x3F800000), jnp.float32
        ) - jnp.float32(1.0)
        q = -jnp.log1p(-uf)
        ratio = probs / q
        return probs, ratio

    rbase = r * V
    vb = jnp.full((SUB, SUBW), neginf, jnp.float32)
    vi = jnp.zeros((SUB, SUBW), jnp.int32)
    vn = jnp.full((SUB, SUBW), big, jnp.int32)
    for k in range(nfull):
        x = logits_ref[rp, :, k * SUBW:(k + 1) * SUBW]
        flat = base_flat + k * SUBW
        probs, ratio = chunk_ratio(x, flat + rbase)
        probs_ref[rp, :, k * SUBW:(k + 1) * SUBW] = probs
        ok = ratio == ratio
        r2 = jnp.where(ok, ratio, neginf)
        upd = r2 > vb
        vb = jnp.maximum(vb, r2)
        vi = jnp.where(upd, flat, vi)
        vn = jnp.minimum(vn, jnp.where(ok, big, flat))
    mx = jnp.max(vb)
    bidx = jnp.min(jnp.where(vb == mx, vi, big))
    nidx = jnp.min(vn)
    if tailw:
        x = logits_ref[rp, :, nfull * SUBW:C]
        flat = tail_flat
        probs, ratio = chunk_ratio(x, flat + rbase)
        probs_ref[rp, :, nfull * SUBW:C] = probs
        ok = ratio == ratio
        r2 = jnp.where(ok, ratio, neginf)
        mxt = jnp.max(r2)
        bt = jnp.min(jnp.where(r2 == mxt, flat, big))
        nt = jnp.min(jnp.where(ok, big, flat))
        bidx = jnp.where(mxt > mx, bt, bidx)
        mx = jnp.maximum(mx, mxt)
        nidx = jnp.minimum(nidx, nt)

    sampled = jnp.where(nidx < big, nidx, bidx)
    toks.append(jnp.where(t_raw < 1e-5, g, sampled))


def kernel(logits, temperatures):
    B, V = logits.shape
    C = V // SUB
    x3 = logits.reshape(B, SUB, C)
    probs3, tok3 = pl.pallas_call(
        functools.partial(_row_body, V=V, C=C),
        grid=(B // RP,),
        in_specs=[
            pl.BlockSpec(memory_space=pltpu.SMEM),
            pl.BlockSpec((RP, SUB, C), lambda r: (r, 0, 0)),
        ],
        out_specs=[
            pl.BlockSpec((RP, SUB, C), lambda r: (r, 0, 0)),
            pl.BlockSpec((RP, 1, 1), lambda r: (r, 0, 0)),
        ],
        out_shape=[
            jax.ShapeDtypeStruct((B, SUB, C), jnp.float32),
            jax.ShapeDtypeStruct((B, 1, 1), jnp.int32),
        ],
    )(temperatures, x3)
    return tok3.reshape(B), probs3.reshape(B, V)


# speedup vs baseline: 1.1573x; 1.1573x over previous
import functools

import jax
import jax.numpy as jnp
from jax.experimental import pallas as pl
from jax.experimental.pallas import tpu as pltpu

SUB = 8
SUBW = 1024


def _rotl(x, d):
    return (x << jnp.uint32(d)) | (x >> jnp.uint32(32 - d))


def _threefry_bits(j):
    ks0 = jnp.uint32(0)
    ks1 = jnp.uint32(1)
    ks2 = jnp.uint32(0x1BD11BDA) ^ ks0 ^ ks1
    ks = (ks0, ks1, ks2)
    rotations = ((13, 15, 26, 6), (17, 29, 16, 24))
    x0 = jnp.zeros_like(j) + ks0
    x1 = j + ks1
    for i in range(5):
        for r in rotations[i % 2]:
            x0 = x0 + x1
            x1 = _rotl(x1, r)
            x1 = x1 ^ x0
        x0 = x0 + ks[(i + 1) % 3]
        x1 = x1 + ks[(i + 2) % 3] + jnp.uint32(i + 1)
    return x0 ^ x1


def _iotas(shape, C):
    sub = jax.lax.broadcasted_iota(jnp.int32, shape, 0)
    lane = jax.lax.broadcasted_iota(jnp.int32, shape, 1)
    return sub * C + lane


def _row_body(temps_ref, logits_ref, probs_ref, tok_ref, *, V, C):
    r = pl.program_id(0)
    t_raw = temps_ref[r]
    t = jnp.where(t_raw < 1e-5, jnp.float32(1.0), t_raw)
    rt = jnp.float32(1.0) / t
    neginf = jnp.float32(-jnp.inf)
    big = jnp.int32(V)

    nfull = C // SUBW
    tailw = C - nfull * SUBW
    base_flat = _iotas((SUB, SUBW), C)
    tail_flat = _iotas((SUB, tailw), C) + nfull * SUBW if tailw else None

    vm = jnp.full((SUB, SUBW), neginf, jnp.float32)
    vg = jnp.full((SUB, SUBW), big, jnp.int32)
    for k in range(nfull):
        sc = logits_ref[0, :, k * SUBW:(k + 1) * SUBW] * rt
        upd = sc > vm
        vm = jnp.maximum(vm, sc)
        vg = jnp.where(upd, base_flat + k * SUBW, vg)
    m = jnp.max(vm)
    g = jnp.min(jnp.where(vm == m, vg, big))
    if tailw:
        sc = logits_ref[0, :, nfull * SUBW:C] * rt
        mt = jnp.max(sc)
        gt = jnp.min(jnp.where(sc == mt, tail_flat, big))
        g = jnp.where(mt > m, gt, g)
        m = jnp.maximum(m, mt)

    vs = jnp.zeros((SUB, SUBW), jnp.float32)
    for k in range(nfull):
        vs = vs + jnp.exp(logits_ref[0, :, k * SUBW:(k + 1) * SUBW] * rt - m)
    s = jnp.sum(vs)
    if tailw:
        s = s + jnp.sum(jnp.exp(logits_ref[0, :, nfull * SUBW:C] * rt - m))
    rs = jnp.float32(1.0) / s

    def chunk_ratio(x, flat0):
        e = jnp.exp(x * rt - m)
        probs = e * rs
        j = flat0.astype(jnp.uint32)
        bits = _threefry_bits(j)
        uf = jax.lax.bitcast_convert_type(
            (bits >> jnp.uint32(9)) | jnp.uint32(0x3F800000), jnp.float32
        ) - jnp.float32(1.0)
        q = -jnp.log1p(-uf)
        ratio = probs / q
        return probs, ratio

    rbase = r * V
    vb = jnp.full((SUB, SUBW), neginf, jnp.float32)
    vi = jnp.zeros((SUB, SUBW), jnp.int32)
    vn = jnp.full((SUB, SUBW), big, jnp.int32)
    for k in range(nfull):
        x = logits_ref[0, :, k * SUBW:(k + 1) * SUBW]
        flat = base_flat + k * SUBW
        probs, ratio = chunk_ratio(x, flat + rbase)
        probs_ref[0, :, k * SUBW:(k + 1) * SUBW] = probs
        ok = ratio == ratio
        r2 = jnp.where(ok, ratio, neginf)
        upd = r2 > vb
        vb = jnp.maximum(vb, r2)
        vi = jnp.where(upd, flat, vi)
        vn = jnp.minimum(vn, jnp.where(ok, big, flat))
    mx = jnp.max(vb)
    bidx = jnp.min(jnp.where(vb == mx, vi, big))
    nidx = jnp.min(vn)
    if tailw:
        x = logits_ref[0, :, nfull * SUBW:C]
        flat = tail_flat
        probs, ratio = chunk_ratio(x, flat + rbase)
        probs_ref[0, :, nfull * SUBW:C] = probs
        ok = ratio == ratio
        r2 = jnp.where(ok, ratio, neginf)
        mxt = jnp.max(r2)
        bt = jnp.min(jnp.where(r2 == mxt, flat, big))
        nt = jnp.min(jnp.where(ok, big, flat))
        bidx = jnp.where(mxt > mx, bt, bidx)
        mx = jnp.maximum(mx, mxt)
        nidx = jnp.minimum(nidx, nt)

    sampled = jnp.where(nidx < big, nidx, bidx)
    tok = jnp.where(t_raw < 1e-5, g, sampled)
    tok_ref[...] = jnp.reshape(tok, (1, 1, 1))


def kernel(logits, temperatures):
    B, V = logits.shape
    C = V // SUB
    x3 = logits.reshape(B, SUB, C)
    probs3, tok3 = pl.pallas_call(
        functools.partial(_row_body, V=V, C=C),
        grid=(B,),
        in_specs=[
            pl.BlockSpec(memory_space=pltpu.SMEM),
            pl.BlockSpec((1, SUB, C), lambda r: (r, 0, 0)),
        ],
        out_specs=[
            pl.BlockSpec((1, SUB, C), lambda r: (r, 0, 0)),
            pl.BlockSpec((1, 1, 1), lambda r: (r, 0, 0)),
        ],
        out_shape=[
            jax.ShapeDtypeStruct((B, SUB, C), jnp.float32),
            jax.ShapeDtypeStruct((B, 1, 1), jnp.int32),
        ],
        compiler_params=pltpu.CompilerParams(
            dimension_semantics=("parallel",),
        ),
    )(temperatures, x3)
    return tok3.reshape(B), probs3.reshape(B, V)


# speedup vs baseline: 1.1578x; 1.0004x over previous
import functools

import jax
import jax.numpy as jnp
from jax.experimental import pallas as pl
from jax.experimental.pallas import tpu as pltpu

SUB = 8
SUBW = 1024


def _rotl(x, d):
    return (x << jnp.uint32(d)) | (x >> jnp.uint32(32 - d))


def _threefry_bits(j):
    ks0 = jnp.uint32(0)
    ks1 = jnp.uint32(1)
    ks2 = jnp.uint32(0x1BD11BDA) ^ ks0 ^ ks1
    ks = (ks0, ks1, ks2)
    rotations = ((13, 15, 26, 6), (17, 29, 16, 24))
    x0 = jnp.zeros_like(j) + ks0
    x1 = j + ks1
    for i in range(5):
        for r in rotations[i % 2]:
            x0 = x0 + x1
            x1 = _rotl(x1, r)
            x1 = x1 ^ x0
        x0 = x0 + ks[(i + 1) % 3]
        x1 = x1 + ks[(i + 2) % 3] + jnp.uint32(i + 1)
    return x0 ^ x1


def _iotas(shape, C):
    sub = jax.lax.broadcasted_iota(jnp.int32, shape, 0)
    lane = jax.lax.broadcasted_iota(jnp.int32, shape, 1)
    return sub * C + lane


def _row_body(temps_ref, logits_ref, probs_ref, tok_ref, *, V, C):
    r = pl.program_id(0)
    t_raw = temps_ref[r]
    t = jnp.where(t_raw < 1e-5, jnp.float32(1.0), t_raw)
    rt = jnp.float32(1.0) / t
    neginf = jnp.float32(-jnp.inf)
    big = jnp.int32(V)

    nfull = C // SUBW
    tailw = C - nfull * SUBW
    base_flat = _iotas((SUB, SUBW), C)
    tail_flat = _iotas((SUB, tailw), C) + nfull * SUBW if tailw else None

    vm = jnp.full((SUB, SUBW), neginf, jnp.float32)
    vg = jnp.full((SUB, SUBW), big, jnp.int32)
    for k in range(nfull):
        sc = logits_ref[:, k * SUBW:(k + 1) * SUBW] * rt
        upd = sc > vm
        vm = jnp.maximum(vm, sc)
        vg = jnp.where(upd, base_flat + k * SUBW, vg)
    m = jnp.max(vm)
    g = jnp.min(jnp.where(vm == m, vg, big))
    if tailw:
        sc = logits_ref[:, nfull * SUBW:C] * rt
        mt = jnp.max(sc)
        gt = jnp.min(jnp.where(sc == mt, tail_flat, big))
        g = jnp.where(mt > m, gt, g)
        m = jnp.maximum(m, mt)

    vs = jnp.zeros((SUB, SUBW), jnp.float32)
    for k in range(nfull):
        vs = vs + jnp.exp(logits_ref[:, k * SUBW:(k + 1) * SUBW] * rt - m)
    s = jnp.sum(vs)
    if tailw:
        s = s + jnp.sum(jnp.exp(logits_ref[:, nfull * SUBW:C] * rt - m))
    rs = jnp.float32(1.0) / s

    def chunk_ratio(x, flat0):
        e = jnp.exp(x * rt - m)
        probs = e * rs
        j = flat0.astype(jnp.uint32)
        bits = _threefry_bits(j)
        uf = jax.lax.bitcast_convert_type(
            (bits >> jnp.uint32(9)) | jnp.uint32(0x3F800000), jnp.float32
        ) - jnp.float32(1.0)
        q = -jnp.log1p(-uf)
        ratio = probs / q
        return probs, ratio

    rbase = r * V
    vb = jnp.full((SUB, SUBW), neginf, jnp.float32)
    vi = jnp.zeros((SUB, SUBW), jnp.int32)
    vn = jnp.full((SUB, SUBW), big, jnp.int32)
    for k in range(nfull):
        x = logits_ref[:, k * SUBW:(k + 1) * SUBW]
        flat = base_flat + k * SUBW
        probs, ratio = chunk_ratio(x, flat + rbase)
        probs_ref[:, k * SUBW:(k + 1) * SUBW] = probs
        ok = ratio == ratio
        r2 = jnp.where(ok, ratio, neginf)
        upd = r2 > vb
        vb = jnp.maximum(vb, r2)
        vi = jnp.where(upd, flat, vi)
        vn = jnp.minimum(vn, jnp.where(ok, big, flat))
    mx = jnp.max(vb)
    bidx = jnp.min(jnp.where(vb == mx, vi, big))
    nidx = jnp.min(vn)
    if tailw:
        x = logits_ref[:, nfull * SUBW:C]
        flat = tail_flat
        probs, ratio = chunk_ratio(x, flat + rbase)
        probs_ref[:, nfull * SUBW:C] = probs
        ok = ratio == ratio
        r2 = jnp.where(ok, ratio, neginf)
        mxt = jnp.max(r2)
        bt = jnp.min(jnp.where(r2 == mxt, flat, big))
        nt = jnp.min(jnp.where(ok, big, flat))
        bidx = jnp.where(mxt > mx, bt, bidx)
        mx = jnp.maximum(mx, mxt)
        nidx = jnp.minimum(nidx, nt)

    sampled = jnp.where(nidx < big, nidx, bidx)
    tok = jnp.where(t_raw < 1e-5, g, sampled)
    tok_ref[...] = jnp.reshape(tok, (1, 1, 1))


def kernel(logits, temperatures):
    B, V = logits.shape
    C = V // SUB
    x2 = logits.reshape(B * SUB, C)
    probs2, tok3 = pl.pallas_call(
        functools.partial(_row_body, V=V, C=C),
        grid=(B,),
        in_specs=[
            pl.BlockSpec(memory_space=pltpu.SMEM),
            pl.BlockSpec((SUB, C), lambda r: (r, 0)),
        ],
        out_specs=[
            pl.BlockSpec((SUB, C), lambda r: (r, 0)),
            pl.BlockSpec((1, 1, 1), lambda r: (r, 0, 0)),
        ],
        out_shape=[
            jax.ShapeDtypeStruct((B * SUB, C), jnp.float32),
            jax.ShapeDtypeStruct((B, 1, 1), jnp.int32),
        ],
        compiler_params=pltpu.CompilerParams(
            dimension_semantics=("parallel",),
        ),
    )(temperatures, x2)
    return tok3.reshape(B), probs2.reshape(B, V)
